# baseline (speedup 1.0000x reference)
import functools

import jax
import jax.numpy as jnp
from jax import lax
from jax.experimental import pallas as pl
from jax.experimental.pallas import tpu as pltpu
from jax.experimental.pallas import tpu_sc as plsc

_DECAY = 0.99


def _make_sc_gather(V, D, B):
    info = plsc.get_sparse_core_info()
    NC, NS = info.num_cores, info.num_subcores
    NW = NC * NS
    CH = 128
    b_per_w = B // NW
    n_ch = b_per_w // CH
    mesh = plsc.VectorSubcoreMesh(core_axis_name="c", subcore_axis_name="s")

    @functools.partial(
        pl.kernel,
        mesh=mesh,
        compiler_params=pltpu.CompilerParams(use_tc_tiling_on_sc=False),
        out_type=jax.ShapeDtypeStruct((B, D), jnp.float32),
        scratch_types=[
            pltpu.VMEM((n_ch, CH), jnp.int32),
            pltpu.VMEM((b_per_w, D), jnp.float32),
            pltpu.SemaphoreType.DMA,
        ],
    )
    def gather_k(mem_hbm, idx_hbm, out_hbm, idx_v, rows_v, sem):
        wid = lax.axis_index("s") * NC + lax.axis_index("c")
        pltpu.sync_copy(idx_hbm.at[pl.ds(wid * n_ch, n_ch)], idx_v)
        cps = []
        for c in range(n_ch):
            cps.append(
                pltpu.async_copy(
                    mem_hbm.at[idx_v.at[c]],
                    rows_v.at[pl.ds(c * CH, CH)],
                    sem,
                )
            )
        for cp in cps:
            cp.wait()
        pltpu.sync_copy(rows_v, out_hbm.at[pl.ds(wid * b_per_w, b_per_w)])

    return gather_k


def _modulate(val, W):
    B, D = val.shape
    BLK = 2048

    def body(val_ref, w_ref, o32_ref, obf_ref):
        w32 = jnp.tanh(
            jnp.dot(val_ref[...], w_ref[...], preferred_element_type=jnp.float32)
        )
        o32_ref[...] = w32
        obf_ref[...] = w32.astype(jnp.bfloat16)

    return pl.pallas_call(
        body,
        grid=(B // BLK,),
        in_specs=[
            pl.BlockSpec((BLK, D), lambda i: (i, 0)),
            pl.BlockSpec((D, D), lambda i: (0, 0)),
        ],
        out_specs=[
            pl.BlockSpec((BLK, D), lambda i: (i, 0)),
            pl.BlockSpec((BLK, D), lambda i: (i, 0)),
        ],
        out_shape=[
            jax.ShapeDtypeStruct((B, D), jnp.float32),
            jax.ShapeDtypeStruct((B, D), jnp.bfloat16),
        ],
    )(val, W)


def _combine(idx2d, g, w32, wbf):
    _, B = idx2d.shape
    D = g.shape[1]
    BLK = 256

    def body(idx_ref, g_ref, w32_ref, wbf_ref, out_ref):
        i = pl.program_id(0)
        idx_all = idx_ref[0, :]
        idx_blk = idx_ref[0, pl.ds(i * BLK, BLK)]
        eq = idx_blk[:, None] == idx_all[None, :]
        mask = eq.astype(jnp.bfloat16)
        dup = lax.dot_general(
            mask,
            wbf_ref[...],
            (((1,), (0,)), ((), ())),
            preferred_element_type=jnp.float32,
        )
        wbf_blk = wbf_ref[pl.ds(i * BLK, BLK), :].astype(jnp.float32)
        out_ref[...] = _DECAY * g_ref[...] + dup + (w32_ref[...] - wbf_blk)

    return pl.pallas_call(
        body,
        grid=(B // BLK,),
        in_specs=[
            pl.BlockSpec((1, B), lambda i: (0, 0)),
            pl.BlockSpec((BLK, D), lambda i: (i, 0)),
            pl.BlockSpec((BLK, D), lambda i: (i, 0)),
            pl.BlockSpec((B, D), lambda i: (0, 0)),
        ],
        out_specs=pl.BlockSpec((BLK, D), lambda i: (i, 0)),
        out_shape=jax.ShapeDtypeStruct((B, D), jnp.float32),
    )(idx2d, g, w32, wbf)


def kernel(mem, idx, val, W_mod):
    V, D = mem.shape
    B = idx.shape[0]
    idx_sc = idx.reshape(-1, 128)
    g = _make_sc_gather(V, D, B)(mem, idx_sc)
    w32, wbf = _modulate(val, W_mod)
    return _combine(idx.reshape(1, B), g, w32, wbf)


# speedup vs baseline: 1.1966x; 1.1966x over previous
import functools

import jax
import jax.numpy as jnp
from jax import lax
from jax.experimental import pallas as pl
from jax.experimental.pallas import tpu as pltpu
from jax.experimental.pallas import tpu_sc as plsc

_DECAY = 0.99
_CH = 128
_SCP = pltpu.CompilerParams(use_tc_tiling_on_sc=False)


def _wid():
    return lax.axis_index("s") * 2 + lax.axis_index("c")


def _mesh():
    return plsc.VectorSubcoreMesh(core_axis_name="c", subcore_axis_name="s")


def _modulate(val, W):
    B, D = val.shape
    BLK = 2048

    def body(val_ref, w_ref, o_ref):
        o_ref[...] = jnp.tanh(
            jnp.dot(val_ref[...], w_ref[...], preferred_element_type=jnp.float32)
        )

    return pl.pallas_call(
        body,
        grid=(B // BLK,),
        in_specs=[
            pl.BlockSpec((BLK, D), lambda i: (i, 0)),
            pl.BlockSpec((D, D), lambda i: (0, 0)),
        ],
        out_specs=pl.BlockSpec((BLK, D), lambda i: (i, 0)),
        out_shape=jax.ShapeDtypeStruct((B, D), jnp.float32),
    )(val, W)


def _make_winner_scatter(V, B, NW, n_ch):
    @functools.partial(
        pl.kernel,
        mesh=_mesh(),
        compiler_params=_SCP,
        out_type=jax.ShapeDtypeStruct((V, 16), jnp.int32),
        scratch_types=[
            pltpu.VMEM((n_ch, _CH), jnp.int32),
            pltpu.VMEM((n_ch * _CH, 16), jnp.int32),
            pltpu.SemaphoreType.DMA,
        ],
    )
    def k(idx_hbm, rid_hbm, s2_hbm, idx_v, rid_v, sem):
        wid = _wid()
        pltpu.sync_copy(idx_hbm.at[pl.ds(wid * n_ch, n_ch)], idx_v)
        pltpu.sync_copy(rid_hbm.at[pl.ds(wid * n_ch * _CH, n_ch * _CH)], rid_v)
        cps = [
            pltpu.async_copy(
                rid_v.at[pl.ds(c * _CH, _CH)], s2_hbm.at[idx_v.at[c]], sem
            )
            for c in range(n_ch)
        ]
        for cp in cps:
            cp.wait()

    return k


def _make_win_gather(V, B, NW, n_ch):
    b_per_w = n_ch * _CH

    @functools.partial(
        pl.kernel,
        mesh=_mesh(),
        compiler_params=_SCP,
        out_type=jax.ShapeDtypeStruct((B, 16), jnp.int32),
        scratch_types=[
            pltpu.VMEM((n_ch, _CH), jnp.int32),
            pltpu.VMEM((b_per_w, 16), jnp.int32),
            pltpu.SemaphoreType.DMA,
        ],
    )
    def k(s2_hbm, idx_hbm, w16_hbm, idx_v, w16_v, sem):
        wid = _wid()
        pltpu.sync_copy(idx_hbm.at[pl.ds(wid * n_ch, n_ch)], idx_v)
        cps = [
            pltpu.async_copy(
                s2_hbm.at[idx_v.at[c]], w16_v.at[pl.ds(c * _CH, _CH)], sem
            )
            for c in range(n_ch)
        ]
        for cp in cps:
            cp.wait()
        pltpu.sync_copy(w16_v, w16_hbm.at[pl.ds(wid * b_per_w, b_per_w)])

    return k


def _make_accumulate(V, B, D, NW, n_ch):
    b_per_w = n_ch * _CH
    rows_per_tile = B // 16

    @functools.partial(
        pl.kernel,
        mesh=_mesh(),
        compiler_params=_SCP,
        out_type=[
            jax.ShapeDtypeStruct((B, D), jnp.float32),
            jax.ShapeDtypeStruct((B, D), jnp.float32),
        ],
        scratch_types=[
            pltpu.VMEM((n_ch, _CH), jnp.int32),
            pltpu.VMEM((b_per_w, D), jnp.float32),
            pltpu.VMEM_SHARED((B, D), jnp.float32),
            pltpu.SemaphoreType.DMA,
        ],
    )
    def k(win_hbm, w_hbm, z_hbm, accA, accB, win_v, w_v, acc_sp, sem):
        cc = lax.axis_index("c")
        ss = lax.axis_index("s")
        wid = ss * 2 + cc
        pltpu.sync_copy(
            z_hbm.at[pl.ds(ss * rows_per_tile, rows_per_tile)],
            acc_sp.at[pl.ds(ss * rows_per_tile, rows_per_tile)],
        )
        pltpu.sync_copy(win_hbm.at[wid], win_v)
        pltpu.sync_copy(w_hbm.at[pl.ds(wid * b_per_w, b_per_w)], w_v)
        plsc.subcore_barrier()
        for c in range(n_ch):
            pltpu.sync_copy(
                w_v.at[pl.ds(c * _CH, _CH)], acc_sp.at[win_v.at[c]], add=True
            )
        plsc.subcore_barrier()

        @pl.when(cc == 0)
        def _():
            pltpu.sync_copy(
                acc_sp.at[pl.ds(ss * rows_per_tile, rows_per_tile)],
                accA.at[pl.ds(ss * rows_per_tile, rows_per_tile)],
            )

        @pl.when(cc == 1)
        def _():
            pltpu.sync_copy(
                acc_sp.at[pl.ds(ss * rows_per_tile, rows_per_tile)],
                accB.at[pl.ds(ss * rows_per_tile, rows_per_tile)],
            )

    return k


def _make_combine(V, B, D, NW, n_ch):
    b_per_w = n_ch * _CH

    @functools.partial(
        pl.kernel,
        mesh=_mesh(),
        compiler_params=_SCP,
        out_type=jax.ShapeDtypeStruct((B, D), jnp.float32),
        scratch_types=[
            pltpu.VMEM((n_ch, _CH), jnp.int32),
            pltpu.VMEM((n_ch, _CH), jnp.int32),
            pltpu.VMEM((b_per_w, D), jnp.float32),
            pltpu.VMEM((b_per_w, D), jnp.float32),
            pltpu.VMEM((b_per_w, D), jnp.float32),
            pltpu.SemaphoreType.DMA,
        ],
    )
    def k(mem_hbm, idx_hbm, win_hbm, accA, accB, out_hbm,
          idx_v, win_v, g_v, a_v, b_v, sem):
        wid = _wid()
        pltpu.sync_copy(idx_hbm.at[pl.ds(wid * n_ch, n_ch)], idx_v)
        pltpu.sync_copy(win_hbm.at[wid], win_v)
        cps = []
        for c in range(n_ch):
            cps.append(
                pltpu.async_copy(
                    mem_hbm.at[idx_v.at[c]], g_v.at[pl.ds(c * _CH, _CH)], sem
                )
            )
            cps.append(
                pltpu.async_copy(
                    accA.at[win_v.at[c]], a_v.at[pl.ds(c * _CH, _CH)], sem
                )
            )
            cps.append(
                pltpu.async_copy(
                    accB.at[win_v.at[c]], b_v.at[pl.ds(c * _CH, _CH)], sem
                )
            )
        for cp in cps:
            cp.wait()

        def row(r, carry):
            for kk in range(D // 16):
                sl = pl.ds(kk * 16, 16)
                g_v[r, sl] = _DECAY * g_v[r, sl] + a_v[r, sl] + b_v[r, sl]
            return carry

        lax.fori_loop(0, b_per_w, row, 0)
        pltpu.sync_copy(g_v, out_hbm.at[pl.ds(wid * b_per_w, b_per_w)])

    return k


def kernel(mem, idx, val, W_mod):
    V, D = mem.shape
    B = idx.shape[0]
    NW = 32
    n_ch = B // (NW * _CH)
    idx2d = idx.reshape(NW * n_ch, _CH)
    rid16 = jnp.broadcast_to(
        jnp.arange(B, dtype=jnp.int32)[:, None], (B, 16)
    )
    zeros = jnp.zeros((B, D), jnp.float32)
    w32 = _modulate(val, W_mod)
    s2 = _make_winner_scatter(V, B, NW, n_ch)(idx2d, rid16)
    win16 = _make_win_gather(V, B, NW, n_ch)(s2, idx2d)
    win = win16[:, 0].reshape(NW, n_ch, _CH)
    accA, accB = _make_accumulate(V, B, D, NW, n_ch)(win, w32, zeros)
    return _make_combine(V, B, D, NW, n_ch)(mem, idx2d, win, accA, accB)
